# Initial kernel scaffold; baseline (speedup 1.0000x reference)
import jax
import jax.numpy as jnp
from jax.experimental import pallas as pl


def kernel(x, edge_index, batch, W1, b1, W2, b2, gamma, beta, Wm1, bm1, Wm2, bm2):
    raise NotImplementedError("write your pallas kernel here")


# speedup vs baseline: 20.0489x; 20.0489x over previous
import jax
import jax.numpy as jnp
from jax import lax
from jax.experimental import pallas as pl
from jax.experimental.pallas import tpu as pltpu
from jax.experimental.pallas import tpu_sc as plsc

N = 10000
E = 320000
D = 128
G = 64

NC = 2
NS = 16
NW = NC * NS
CH = 128
KJ = 80
EP = NW * KJ * CH
NACC = 10240
RPS = NACC // NS
DW = 16

_sc_mesh = plsc.VectorSubcoreMesh(core_axis_name="c", subcore_axis_name="s")


def _sc_deg_body(dst_hbm, zeros_hbm, out_hbm, acc, dst_buf, ones_v):
    c = lax.axis_index("c")
    s = lax.axis_index("s")
    w = c * NS + s
    pltpu.sync_copy(zeros_hbm.at[pl.ds(s * RPS, RPS)], acc.at[pl.ds(s * RPS, RPS)])
    pltpu.sync_copy(dst_hbm.at[pl.ds(w * KJ, KJ)], dst_buf)

    def fill_ones(i, carry):
        for b in range(D // 16):
            ones_v[i, pl.ds(b * 16, 16)] = jnp.ones((16,), jnp.float32)
        return carry

    lax.fori_loop(0, CH, fill_ones, 0)
    plsc.subcore_barrier()

    def body(j, carry):
        pltpu.sync_copy(ones_v, acc.at[dst_buf.at[j]], add=True)
        return carry

    lax.fori_loop(0, KJ, body, 0)
    plsc.subcore_barrier()
    for k in range(RPS // CH):
        base = s * RPS + k * CH
        pltpu.sync_copy(acc.at[pl.ds(base, CH)], ones_v)
        pltpu.sync_copy(ones_v, out_hbm.at[c, pl.ds(base, CH)])


_deg_call = pl.kernel(
    _sc_deg_body,
    out_type=jax.ShapeDtypeStruct((NC, NACC, D), jnp.float32),
    mesh=_sc_mesh,
    scratch_types=[
        pltpu.VMEM_SHARED((NACC, D), jnp.float32),
        pltpu.VMEM((KJ, CH), jnp.int32),
        pltpu.VMEM((CH, D), jnp.float32),
    ],
)


def _sc_agg_body(src_hbm, dst_hbm, g_hbm, zeros_hbm, out_hbm,
                 acc, src_buf, dst_buf, rows_v, sem):
    c = lax.axis_index("c")
    s = lax.axis_index("s")
    w = c * NS + s
    pltpu.sync_copy(zeros_hbm.at[pl.ds(s * RPS, RPS)], acc.at[pl.ds(s * RPS, RPS)])
    pltpu.sync_copy(src_hbm.at[pl.ds(w * KJ, KJ)], src_buf)
    pltpu.sync_copy(dst_hbm.at[pl.ds(w * KJ, KJ)], dst_buf)
    plsc.subcore_barrier()

    def body(j, carry):
        pltpu.async_copy(g_hbm.at[src_buf.at[j]], rows_v, sem).wait()
        pltpu.sync_copy(rows_v, acc.at[dst_buf.at[j]], add=True)
        return carry

    lax.fori_loop(0, KJ, body, 0)
    plsc.subcore_barrier()
    for k in range(RPS // CH):
        base = s * RPS + k * CH
        pltpu.sync_copy(acc.at[pl.ds(base, CH)], rows_v)
        pltpu.sync_copy(rows_v, out_hbm.at[c, pl.ds(base, CH)])


_agg_call = pl.kernel(
    _sc_agg_body,
    out_type=jax.ShapeDtypeStruct((NC, NACC, D), jnp.float32),
    mesh=_sc_mesh,
    scratch_types=[
        pltpu.VMEM_SHARED((NACC, D), jnp.float32),
        pltpu.VMEM((KJ, CH), jnp.int32),
        pltpu.VMEM((KJ, CH), jnp.int32),
        pltpu.VMEM((CH, D), jnp.float32),
        pltpu.SemaphoreType.DMA,
    ],
)


BLK = 1000


def _tc1_body(degp_ref, x_ref, w1_ref, g1_ref):
    deg = degp_ref[0, :, 0:1] + degp_ref[1, :, 0:1] + 1.0
    dis = lax.rsqrt(deg)
    g1_ref[...] = jnp.dot(x_ref[...], w1_ref[...],
                          preferred_element_type=jnp.float32) * dis


def _tc1(degp, x, W1):
    return pl.pallas_call(
        _tc1_body,
        grid=(N // BLK,),
        in_specs=[
            pl.BlockSpec((NC, BLK, D), lambda i: (0, i, 0)),
            pl.BlockSpec((BLK, D), lambda i: (i, 0)),
            pl.BlockSpec((D, D), lambda i: (0, 0)),
        ],
        out_specs=pl.BlockSpec((BLK, D), lambda i: (i, 0)),
        out_shape=jax.ShapeDtypeStruct((N, D), jnp.float32),
    )(degp, x, W1)


def _tc2_body(degp_ref, ap_ref, g1_ref, w2_ref, b1_ref, g2_ref):
    deg = degp_ref[0, :, 0:1] + degp_ref[1, :, 0:1] + 1.0
    dis = lax.rsqrt(deg)
    h1 = jnp.maximum((ap_ref[0] + ap_ref[1] + g1_ref[...]) * dis + b1_ref[...], 0.0)
    g2_ref[...] = jnp.dot(h1, w2_ref[...],
                          preferred_element_type=jnp.float32) * dis


def _tc2(degp, a1, g1, W2, b1):
    return pl.pallas_call(
        _tc2_body,
        grid=(N // BLK,),
        in_specs=[
            pl.BlockSpec((NC, BLK, D), lambda i: (0, i, 0)),
            pl.BlockSpec((NC, BLK, D), lambda i: (0, i, 0)),
            pl.BlockSpec((BLK, D), lambda i: (i, 0)),
            pl.BlockSpec((D, D), lambda i: (0, 0)),
            pl.BlockSpec((1, D), lambda i: (0, 0)),
        ],
        out_specs=pl.BlockSpec((BLK, D), lambda i: (i, 0)),
        out_shape=jax.ShapeDtypeStruct((N, D), jnp.float32),
    )(degp, a1, g1, W2, b1)


def _tc3_body(degp_ref, ap_ref, g2_ref, b2_ref, gam_ref, bet_ref, batch_ref,
              wm1_ref, bm1_ref, wm2_ref, bm2_ref, out_ref):
    deg = degp_ref[0, :N, 0:1] + degp_ref[1, :N, 0:1] + 1.0
    dis = lax.rsqrt(deg)
    h2 = (ap_ref[0, :N] + ap_ref[1, :N] + g2_ref[...]) * dis + b2_ref[...]
    mean = jnp.mean(h2, axis=0, keepdims=True)
    var = jnp.mean((h2 - mean) ** 2, axis=0, keepdims=True)
    hb = (h2 - mean) * lax.rsqrt(var + 1e-5) * gam_ref[...] + bet_ref[...]
    ohT = (jax.lax.broadcasted_iota(jnp.int32, (G, N), 0)
           == batch_ref[...]).astype(jnp.float32)
    sums = jnp.dot(ohT, hb, preferred_element_type=jnp.float32)
    counts = jnp.sum(ohT, axis=1, keepdims=True)
    pooled = sums / jnp.maximum(counts, 1.0)
    z = jnp.maximum(jnp.dot(pooled, wm1_ref[...],
                            preferred_element_type=jnp.float32) + bm1_ref[...], 0.0)
    out_ref[...] = jnp.dot(z, wm2_ref[...],
                           preferred_element_type=jnp.float32) + bm2_ref[...]


def _tc3(degp, a2, g2, b2, gamma, beta, batch_row, Wm1, bm1, Wm2, bm2):
    return pl.pallas_call(
        _tc3_body,
        in_specs=[
            pl.BlockSpec((NC, NACC, D), lambda: (0, 0, 0)),
            pl.BlockSpec((NC, NACC, D), lambda: (0, 0, 0)),
            pl.BlockSpec((N, D), lambda: (0, 0)),
            pl.BlockSpec((1, D), lambda: (0, 0)),
            pl.BlockSpec((1, D), lambda: (0, 0)),
            pl.BlockSpec((1, D), lambda: (0, 0)),
            pl.BlockSpec((1, N), lambda: (0, 0)),
            pl.BlockSpec((D, D), lambda: (0, 0)),
            pl.BlockSpec((1, D), lambda: (0, 0)),
            pl.BlockSpec((D, 1), lambda: (0, 0)),
            pl.BlockSpec((1, 1), lambda: (0, 0)),
        ],
        out_specs=pl.BlockSpec((G, 1), lambda: (0, 0)),
        out_shape=jax.ShapeDtypeStruct((G, 1), jnp.float32),
    )(degp, a2, g2, b2, gamma, beta, batch_row, Wm1, bm1, Wm2, bm2)


def kernel(x, edge_index, batch, W1, b1, W2, b2, gamma, beta, Wm1, bm1, Wm2, bm2):
    src = edge_index[0]
    dst = edge_index[1]
    pad = EP - E
    ar = jnp.arange(pad, dtype=jnp.int32)
    pad_src = ar % N
    pad_dst = N + (ar % (NACC - N))
    srcp = jnp.concatenate([src, pad_src]).reshape(NW * KJ, CH)
    dstp = jnp.concatenate([dst, pad_dst]).reshape(NW * KJ, CH)
    zeros_acc = jnp.zeros((NACC, D), jnp.float32)

    degp = _deg_call(dstp, zeros_acc)
    g1 = _tc1(degp, x, W1)
    a1 = _agg_call(srcp, dstp, g1, zeros_acc)
    g2 = _tc2(degp, a1, g1, W2, b1.reshape(1, D))
    a2 = _agg_call(srcp, dstp, g2, zeros_acc)
    return _tc3(degp, a2, g2, b2.reshape(1, D), gamma.reshape(1, D),
                beta.reshape(1, D), batch.reshape(1, N), Wm1,
                bm1.reshape(1, D), Wm2, bm2.reshape(1, 1))


# speedup vs baseline: 21.0194x; 1.0484x over previous
import jax
import jax.numpy as jnp
from jax import lax
from jax.experimental import pallas as pl
from jax.experimental.pallas import tpu as pltpu
from jax.experimental.pallas import tpu_sc as plsc

N = 10000
E = 320000
D = 128
G = 64

NC = 2
NS = 16
NW = NC * NS
CH = 128
KJ = 80
EP = NW * KJ * CH
NACC = 10240
RPS = NACC // NS
DW = 16

_sc_mesh = plsc.VectorSubcoreMesh(core_axis_name="c", subcore_axis_name="s")


def _sc_deg_body(dst_hbm, zeros_hbm, out_hbm, acc, dst_buf, ones_v, sem):
    c = lax.axis_index("c")
    s = lax.axis_index("s")
    w = c * NS + s
    pltpu.sync_copy(zeros_hbm.at[pl.ds(s * RPS, RPS)], acc.at[pl.ds(s * RPS, RPS)])
    pltpu.sync_copy(dst_hbm.at[pl.ds(w * KJ, KJ)], dst_buf)

    def fill_ones(i, carry):
        for b in range(D // 16):
            ones_v[i, pl.ds(b * 16, 16)] = jnp.ones((16,), jnp.float32)
        return carry

    lax.fori_loop(0, CH, fill_ones, 0)
    plsc.subcore_barrier()

    def body(jj, carry):
        j0 = jj * 8
        for b in range(8):
            pltpu.async_copy(ones_v, acc.at[dst_buf.at[j0 + b]], sem, add=True)
        for b in range(8):
            pltpu.make_async_copy(ones_v, acc.at[dst_buf.at[j0 + b]], sem).wait()
        return carry

    lax.fori_loop(0, KJ // 8, body, 0)
    plsc.subcore_barrier()
    for k in range(RPS // CH):
        base = s * RPS + k * CH
        pltpu.sync_copy(acc.at[pl.ds(base, CH)], ones_v)
        pltpu.sync_copy(ones_v, out_hbm.at[c, pl.ds(base, CH)])


_deg_call = pl.kernel(
    _sc_deg_body,
    out_type=jax.ShapeDtypeStruct((NC, NACC, D), jnp.float32),
    mesh=_sc_mesh,
    scratch_types=[
        pltpu.VMEM_SHARED((NACC, D), jnp.float32),
        pltpu.VMEM((KJ, CH), jnp.int32),
        pltpu.VMEM((CH, D), jnp.float32),
        pltpu.SemaphoreType.DMA,
    ],
)


NB = 2
CHA = 64
KJA = EP // (NW * CHA)
HK = KJA // 2


def _sc_agg_body(src_hbm, dst_hbm, g_hbm, zeros_hbm, out_hbm,
                 acc, src_buf, dst_buf, rows0, rows1,
                 sg0, sg1, ss0, ss1):
    c = lax.axis_index("c")
    s = lax.axis_index("s")
    w = c * NS + s
    pltpu.sync_copy(zeros_hbm.at[pl.ds(s * RPS, RPS)], acc.at[pl.ds(s * RPS, RPS)])
    rows = (rows0, rows1)
    sgs = (sg0, sg1)
    sss = (ss0, ss1)
    for phase in range(KJA // HK):
        base_j = w * KJA + phase * HK
        pltpu.sync_copy(src_hbm.at[pl.ds(base_j, HK)], src_buf)
        pltpu.sync_copy(dst_hbm.at[pl.ds(base_j, HK)], dst_buf)
        if phase == 0:
            plsc.subcore_barrier()
        for b in range(NB):
            pltpu.async_copy(g_hbm.at[src_buf.at[b]], rows[b], sgs[b])

        def body(jj, carry):
            j0 = jj * NB
            for b in range(NB):
                jb = j0 + b
                pltpu.make_async_copy(g_hbm.at[src_buf.at[jb]], rows[b], sgs[b]).wait()
                pltpu.async_copy(rows[b], acc.at[dst_buf.at[jb]], sss[b], add=True)
            for b in range(NB):
                jb = j0 + b
                pltpu.make_async_copy(rows[b], acc.at[dst_buf.at[jb]], sss[b]).wait()

                @pl.when(jb + NB < HK)
                def _():
                    pltpu.async_copy(g_hbm.at[src_buf.at[jb + NB]], rows[b], sgs[b])

            return carry

        lax.fori_loop(0, HK // NB, body, 0)
    plsc.subcore_barrier()
    for k in range(RPS // CHA):
        base = s * RPS + k * CHA
        pltpu.sync_copy(acc.at[pl.ds(base, CHA)], rows0)
        pltpu.sync_copy(rows0, out_hbm.at[c, pl.ds(base, CHA)])


_agg_call = pl.kernel(
    _sc_agg_body,
    out_type=jax.ShapeDtypeStruct((NC, NACC, D), jnp.float32),
    mesh=_sc_mesh,
    scratch_types=[
        pltpu.VMEM_SHARED((NACC, D), jnp.float32),
        pltpu.VMEM((HK, CHA), jnp.int32),
        pltpu.VMEM((HK, CHA), jnp.int32),
        pltpu.VMEM((CHA, D), jnp.float32),
        pltpu.VMEM((CHA, D), jnp.float32),
        pltpu.SemaphoreType.DMA,
        pltpu.SemaphoreType.DMA,
        pltpu.SemaphoreType.DMA,
        pltpu.SemaphoreType.DMA,
    ],
)


BLK = 1000


def _tc1_body(degp_ref, x_ref, w1_ref, g1_ref):
    deg = degp_ref[0, :, 0:1] + degp_ref[1, :, 0:1] + 1.0
    dis = lax.rsqrt(deg)
    g1_ref[...] = jnp.dot(x_ref[...], w1_ref[...],
                          preferred_element_type=jnp.float32) * dis


def _tc1(degp, x, W1):
    return pl.pallas_call(
        _tc1_body,
        grid=(N // BLK,),
        in_specs=[
            pl.BlockSpec((NC, BLK, D), lambda i: (0, i, 0)),
            pl.BlockSpec((BLK, D), lambda i: (i, 0)),
            pl.BlockSpec((D, D), lambda i: (0, 0)),
        ],
        out_specs=pl.BlockSpec((BLK, D), lambda i: (i, 0)),
        out_shape=jax.ShapeDtypeStruct((N, D), jnp.float32),
    )(degp, x, W1)


def _tc2_body(degp_ref, ap_ref, g1_ref, w2_ref, b1_ref, g2_ref):
    deg = degp_ref[0, :, 0:1] + degp_ref[1, :, 0:1] + 1.0
    dis = lax.rsqrt(deg)
    h1 = jnp.maximum((ap_ref[0] + ap_ref[1] + g1_ref[...]) * dis + b1_ref[...], 0.0)
    g2_ref[...] = jnp.dot(h1, w2_ref[...],
                          preferred_element_type=jnp.float32) * dis


def _tc2(degp, a1, g1, W2, b1):
    return pl.pallas_call(
        _tc2_body,
        grid=(N // BLK,),
        in_specs=[
            pl.BlockSpec((NC, BLK, D), lambda i: (0, i, 0)),
            pl.BlockSpec((NC, BLK, D), lambda i: (0, i, 0)),
            pl.BlockSpec((BLK, D), lambda i: (i, 0)),
            pl.BlockSpec((D, D), lambda i: (0, 0)),
            pl.BlockSpec((1, D), lambda i: (0, 0)),
        ],
        out_specs=pl.BlockSpec((BLK, D), lambda i: (i, 0)),
        out_shape=jax.ShapeDtypeStruct((N, D), jnp.float32),
    )(degp, a1, g1, W2, b1)


def _tc3_body(degp_ref, ap_ref, g2_ref, b2_ref, gam_ref, bet_ref, batch_ref,
              wm1_ref, bm1_ref, wm2_ref, bm2_ref, out_ref):
    deg = degp_ref[0, :N, 0:1] + degp_ref[1, :N, 0:1] + 1.0
    dis = lax.rsqrt(deg)
    h2 = (ap_ref[0, :N] + ap_ref[1, :N] + g2_ref[...]) * dis + b2_ref[...]
    mean = jnp.mean(h2, axis=0, keepdims=True)
    var = jnp.mean((h2 - mean) ** 2, axis=0, keepdims=True)
    hb = (h2 - mean) * lax.rsqrt(var + 1e-5) * gam_ref[...] + bet_ref[...]
    ohT = (jax.lax.broadcasted_iota(jnp.int32, (G, N), 0)
           == batch_ref[...]).astype(jnp.float32)
    sums = jnp.dot(ohT, hb, preferred_element_type=jnp.float32)
    counts = jnp.sum(ohT, axis=1, keepdims=True)
    pooled = sums / jnp.maximum(counts, 1.0)
    z = jnp.maximum(jnp.dot(pooled, wm1_ref[...],
                            preferred_element_type=jnp.float32) + bm1_ref[...], 0.0)
    out_ref[...] = jnp.dot(z, wm2_ref[...],
                           preferred_element_type=jnp.float32) + bm2_ref[...]


def _tc3(degp, a2, g2, b2, gamma, beta, batch_row, Wm1, bm1, Wm2, bm2):
    return pl.pallas_call(
        _tc3_body,
        in_specs=[
            pl.BlockSpec((NC, NACC, D), lambda: (0, 0, 0)),
            pl.BlockSpec((NC, NACC, D), lambda: (0, 0, 0)),
            pl.BlockSpec((N, D), lambda: (0, 0)),
            pl.BlockSpec((1, D), lambda: (0, 0)),
            pl.BlockSpec((1, D), lambda: (0, 0)),
            pl.BlockSpec((1, D), lambda: (0, 0)),
            pl.BlockSpec((1, N), lambda: (0, 0)),
            pl.BlockSpec((D, D), lambda: (0, 0)),
            pl.BlockSpec((1, D), lambda: (0, 0)),
            pl.BlockSpec((D, 1), lambda: (0, 0)),
            pl.BlockSpec((1, 1), lambda: (0, 0)),
        ],
        out_specs=pl.BlockSpec((G, 1), lambda: (0, 0)),
        out_shape=jax.ShapeDtypeStruct((G, 1), jnp.float32),
    )(degp, a2, g2, b2, gamma, beta, batch_row, Wm1, bm1, Wm2, bm2)


def kernel(x, edge_index, batch, W1, b1, W2, b2, gamma, beta, Wm1, bm1, Wm2, bm2):
    src = edge_index[0]
    dst = edge_index[1]
    pad = EP - E
    ar = jnp.arange(pad, dtype=jnp.int32)
    pad_src = ar % N
    pad_dst = N + (ar % (NACC - N))
    srcp = jnp.concatenate([src, pad_src]).reshape(NW * KJ, CH)
    dstp = jnp.concatenate([dst, pad_dst]).reshape(NW * KJ, CH)
    srcp64 = srcp.reshape(NW * KJA, CHA)
    dstp64 = dstp.reshape(NW * KJA, CHA)
    zeros_acc = jnp.zeros((NACC, D), jnp.float32)

    degp = _deg_call(dstp, zeros_acc)
    g1 = _tc1(degp, x, W1)
    a1 = _agg_call(srcp64, dstp64, g1, zeros_acc)
    g2 = _tc2(degp, a1, g1, W2, b1.reshape(1, D))
    a2 = _agg_call(srcp64, dstp64, g2, zeros_acc)
    return _tc3(degp, a2, g2, b2.reshape(1, D), gamma.reshape(1, D),
                beta.reshape(1, D), batch.reshape(1, N), Wm1,
                bm1.reshape(1, D), Wm2, bm2.reshape(1, 1))


# speedup vs baseline: 23.2469x; 1.1060x over previous
import jax
import jax.numpy as jnp
from jax import lax
from jax.experimental import pallas as pl
from jax.experimental.pallas import tpu as pltpu
from jax.experimental.pallas import tpu_sc as plsc

N = 10000
E = 320000
D = 128
G = 64

NC = 2
NS = 16
NW = NC * NS
CH = 128
KJ = 80
EP = NW * KJ * CH
NACC = 10240
RPS = NACC // NS
DW = 16

_sc_mesh = plsc.VectorSubcoreMesh(core_axis_name="c", subcore_axis_name="s")


def _sc_deg_body(dst_hbm, zeros_hbm, out_hbm, acc1d, dst_buf, ones_v, tmp_v, sem):
    c = lax.axis_index("c")
    s = lax.axis_index("s")
    w = c * NS + s
    pltpu.sync_copy(zeros_hbm.at[pl.ds(s * RPS, RPS)], acc1d.at[pl.ds(s * RPS, RPS)])
    pltpu.sync_copy(dst_hbm.at[pl.ds(w * KJ, KJ)], dst_buf)

    def fill_ones(i, carry):
        ones_v[pl.ds(i * 16, 16)] = jnp.ones((16,), jnp.float32)
        return carry

    lax.fori_loop(0, CH // 16, fill_ones, 0)
    plsc.subcore_barrier()

    def body(jj, carry):
        j0 = jj * 8
        for b in range(8):
            pltpu.async_copy(ones_v, acc1d.at[dst_buf.at[j0 + b]], sem, add=True)
        for b in range(8):
            pltpu.make_async_copy(ones_v, acc1d.at[dst_buf.at[j0 + b]], sem).wait()
        return carry

    lax.fori_loop(0, KJ // 8, body, 0)
    plsc.subcore_barrier()
    pltpu.sync_copy(acc1d.at[pl.ds(s * RPS, RPS)], tmp_v)
    pltpu.sync_copy(tmp_v, out_hbm.at[pl.ds(c * NACC + s * RPS, RPS)])


_deg_call = pl.kernel(
    _sc_deg_body,
    out_type=jax.ShapeDtypeStruct((NC * NACC,), jnp.float32),
    mesh=_sc_mesh,
    scratch_types=[
        pltpu.VMEM_SHARED((NACC,), jnp.float32),
        pltpu.VMEM((KJ, CH), jnp.int32),
        pltpu.VMEM((CH,), jnp.float32),
        pltpu.VMEM((RPS,), jnp.float32),
        pltpu.SemaphoreType.DMA,
    ],
)


NB = 2
CHA = 64
KJA = EP // (NW * CHA)
HK = KJA // 2


def _sc_agg_body(src_hbm, dst_hbm, g_hbm, zeros_hbm, out_hbm,
                 acc, src_buf, dst_buf, rows0, rows1,
                 sg0, sg1, ss0, ss1):
    c = lax.axis_index("c")
    s = lax.axis_index("s")
    w = c * NS + s
    pltpu.sync_copy(zeros_hbm.at[pl.ds(s * RPS, RPS)], acc.at[pl.ds(s * RPS, RPS)])
    rows = (rows0, rows1)
    sgs = (sg0, sg1)
    sss = (ss0, ss1)
    for phase in range(KJA // HK):
        base_j = w * KJA + phase * HK
        pltpu.sync_copy(src_hbm.at[pl.ds(base_j, HK)], src_buf)
        pltpu.sync_copy(dst_hbm.at[pl.ds(base_j, HK)], dst_buf)
        if phase == 0:
            plsc.subcore_barrier()
        for b in range(NB):
            pltpu.async_copy(g_hbm.at[src_buf.at[b]], rows[b], sgs[b])

        def body(jj, carry):
            j0 = jj * NB
            for b in range(NB):
                jb = j0 + b
                pltpu.make_async_copy(g_hbm.at[src_buf.at[jb]], rows[b], sgs[b]).wait()
                pltpu.async_copy(rows[b], acc.at[dst_buf.at[jb]], sss[b], add=True)
            for b in range(NB):
                jb = j0 + b
                pltpu.make_async_copy(rows[b], acc.at[dst_buf.at[jb]], sss[b]).wait()

                @pl.when(jb + NB < HK)
                def _():
                    pltpu.async_copy(g_hbm.at[src_buf.at[jb + NB]], rows[b], sgs[b])

            return carry

        lax.fori_loop(0, HK // NB, body, 0)
    plsc.subcore_barrier()
    for k in range(RPS // CHA):
        base = s * RPS + k * CHA
        pltpu.sync_copy(acc.at[pl.ds(base, CHA)], rows0)
        pltpu.sync_copy(rows0, out_hbm.at[c, pl.ds(base, CHA)])


_agg_call = pl.kernel(
    _sc_agg_body,
    out_type=jax.ShapeDtypeStruct((NC, NACC, D), jnp.float32),
    mesh=_sc_mesh,
    scratch_types=[
        pltpu.VMEM_SHARED((NACC, D), jnp.float32),
        pltpu.VMEM((HK, CHA), jnp.int32),
        pltpu.VMEM((HK, CHA), jnp.int32),
        pltpu.VMEM((CHA, D), jnp.float32),
        pltpu.VMEM((CHA, D), jnp.float32),
        pltpu.SemaphoreType.DMA,
        pltpu.SemaphoreType.DMA,
        pltpu.SemaphoreType.DMA,
        pltpu.SemaphoreType.DMA,
    ],
)


BLK = 1000


def _tc1_body(degp_ref, x_ref, w1_ref, g1_ref):
    deg = degp_ref[0, :, 0:1] + degp_ref[1, :, 0:1] + 1.0
    dis = lax.rsqrt(deg)
    g1_ref[...] = jnp.dot(x_ref[...], w1_ref[...],
                          preferred_element_type=jnp.float32) * dis


def _tc1(degp, x, W1):
    return pl.pallas_call(
        _tc1_body,
        grid=(N // BLK,),
        in_specs=[
            pl.BlockSpec((NC, BLK, 1), lambda i: (0, i, 0)),
            pl.BlockSpec((BLK, D), lambda i: (i, 0)),
            pl.BlockSpec((D, D), lambda i: (0, 0)),
        ],
        out_specs=pl.BlockSpec((BLK, D), lambda i: (i, 0)),
        out_shape=jax.ShapeDtypeStruct((N, D), jnp.float32),
    )(degp, x, W1)


def _tc2_body(degp_ref, ap_ref, g1_ref, w2_ref, b1_ref, g2_ref):
    deg = degp_ref[0, :, 0:1] + degp_ref[1, :, 0:1] + 1.0
    dis = lax.rsqrt(deg)
    h1 = jnp.maximum((ap_ref[0] + ap_ref[1] + g1_ref[...]) * dis + b1_ref[...], 0.0)
    g2_ref[...] = jnp.dot(h1, w2_ref[...],
                          preferred_element_type=jnp.float32) * dis


def _tc2(degp, a1, g1, W2, b1):
    return pl.pallas_call(
        _tc2_body,
        grid=(N // BLK,),
        in_specs=[
            pl.BlockSpec((NC, BLK, 1), lambda i: (0, i, 0)),
            pl.BlockSpec((NC, BLK, D), lambda i: (0, i, 0)),
            pl.BlockSpec((BLK, D), lambda i: (i, 0)),
            pl.BlockSpec((D, D), lambda i: (0, 0)),
            pl.BlockSpec((1, D), lambda i: (0, 0)),
        ],
        out_specs=pl.BlockSpec((BLK, D), lambda i: (i, 0)),
        out_shape=jax.ShapeDtypeStruct((N, D), jnp.float32),
    )(degp, a1, g1, W2, b1)


def _tc3_body(degp_ref, ap_ref, g2_ref, b2_ref, gam_ref, bet_ref, batch_ref,
              wm1_ref, bm1_ref, wm2_ref, bm2_ref, out_ref):
    deg = degp_ref[0, :N, 0:1] + degp_ref[1, :N, 0:1] + 1.0
    dis = lax.rsqrt(deg)
    h2 = (ap_ref[0, :N] + ap_ref[1, :N] + g2_ref[...]) * dis + b2_ref[...]
    mean = jnp.mean(h2, axis=0, keepdims=True)
    var = jnp.mean((h2 - mean) ** 2, axis=0, keepdims=True)
    hb = (h2 - mean) * lax.rsqrt(var + 1e-5) * gam_ref[...] + bet_ref[...]
    ohT = (jax.lax.broadcasted_iota(jnp.int32, (G, N), 0)
           == batch_ref[...]).astype(jnp.float32)
    sums = jnp.dot(ohT, hb, preferred_element_type=jnp.float32)
    counts = jnp.sum(ohT, axis=1, keepdims=True)
    pooled = sums / jnp.maximum(counts, 1.0)
    z = jnp.maximum(jnp.dot(pooled, wm1_ref[...],
                            preferred_element_type=jnp.float32) + bm1_ref[...], 0.0)
    out_ref[...] = jnp.dot(z, wm2_ref[...],
                           preferred_element_type=jnp.float32) + bm2_ref[...]


def _tc3(degp, a2, g2, b2, gamma, beta, batch_row, Wm1, bm1, Wm2, bm2):
    return pl.pallas_call(
        _tc3_body,
        in_specs=[
            pl.BlockSpec((NC, NACC, 1), lambda: (0, 0, 0)),
            pl.BlockSpec((NC, NACC, D), lambda: (0, 0, 0)),
            pl.BlockSpec((N, D), lambda: (0, 0)),
            pl.BlockSpec((1, D), lambda: (0, 0)),
            pl.BlockSpec((1, D), lambda: (0, 0)),
            pl.BlockSpec((1, D), lambda: (0, 0)),
            pl.BlockSpec((1, N), lambda: (0, 0)),
            pl.BlockSpec((D, D), lambda: (0, 0)),
            pl.BlockSpec((1, D), lambda: (0, 0)),
            pl.BlockSpec((D, 1), lambda: (0, 0)),
            pl.BlockSpec((1, 1), lambda: (0, 0)),
        ],
        out_specs=pl.BlockSpec((G, 1), lambda: (0, 0)),
        out_shape=jax.ShapeDtypeStruct((G, 1), jnp.float32),
    )(degp, a2, g2, b2, gamma, beta, batch_row, Wm1, bm1, Wm2, bm2)


def kernel(x, edge_index, batch, W1, b1, W2, b2, gamma, beta, Wm1, bm1, Wm2, bm2):
    src = edge_index[0]
    dst = edge_index[1]
    pad = EP - E
    ar = jnp.arange(pad, dtype=jnp.int32)
    pad_src = ar % N
    pad_dst = N + (ar % (NACC - N))
    srcp = jnp.concatenate([src, pad_src]).reshape(NW * KJ, CH)
    dstp = jnp.concatenate([dst, pad_dst]).reshape(NW * KJ, CH)
    srcp64 = srcp.reshape(NW * KJA, CHA)
    dstp64 = dstp.reshape(NW * KJA, CHA)
    zeros_acc = jnp.zeros((NACC, D), jnp.float32)

    zeros1d = jnp.zeros((NACC,), jnp.float32)
    degp = _deg_call(dstp, zeros1d).reshape(NC, NACC, 1)
    g1 = _tc1(degp, x, W1)
    a1 = _agg_call(srcp64, dstp64, g1, zeros_acc)
    g2 = _tc2(degp, a1, g1, W2, b1.reshape(1, D))
    a2 = _agg_call(srcp64, dstp64, g2, zeros_acc)
    return _tc3(degp, a2, g2, b2.reshape(1, D), gamma.reshape(1, D),
                beta.reshape(1, D), batch.reshape(1, N), Wm1,
                bm1.reshape(1, D), Wm2, bm2.reshape(1, 1))


# speedup vs baseline: 23.9746x; 1.0313x over previous
import jax
import jax.numpy as jnp
from jax import lax
from jax.experimental import pallas as pl
from jax.experimental.pallas import tpu as pltpu
from jax.experimental.pallas import tpu_sc as plsc

N = 10000
E = 320000
D = 128
G = 64

NC = 2
NS = 16
NW = NC * NS
CH = 128
KJ = 80
EP = NW * KJ * CH
NACC = 10240
RPS = NACC // NS
DW = 16

_sc_mesh = plsc.VectorSubcoreMesh(core_axis_name="c", subcore_axis_name="s")


def _sc_deg_body(dst_hbm, out_hbm, acc1d, dst_buf, ones_v, tmp_v, sem):
    c = lax.axis_index("c")
    s = lax.axis_index("s")
    w = c * NS + s

    def ztmp(i, carry):
        tmp_v[pl.ds(i * 16, 16)] = jnp.zeros((16,), jnp.float32)
        return carry

    lax.fori_loop(0, RPS // 16, ztmp, 0)
    pltpu.sync_copy(tmp_v, acc1d.at[pl.ds(s * RPS, RPS)])
    pltpu.sync_copy(dst_hbm.at[pl.ds(w * KJ, KJ)], dst_buf)

    def fill_ones(i, carry):
        ones_v[pl.ds(i * 16, 16)] = jnp.ones((16,), jnp.float32)
        return carry

    lax.fori_loop(0, CH // 16, fill_ones, 0)
    plsc.subcore_barrier()

    def body(jj, carry):
        j0 = jj * 8
        for b in range(8):
            pltpu.async_copy(ones_v, acc1d.at[dst_buf.at[j0 + b]], sem, add=True)
        for b in range(8):
            pltpu.make_async_copy(ones_v, acc1d.at[dst_buf.at[j0 + b]], sem).wait()
        return carry

    lax.fori_loop(0, KJ // 8, body, 0)
    plsc.subcore_barrier()
    pltpu.sync_copy(acc1d.at[pl.ds(s * RPS, RPS)], tmp_v)
    pltpu.sync_copy(tmp_v, out_hbm.at[pl.ds(c * NACC + s * RPS, RPS)])


_deg_call = pl.kernel(
    _sc_deg_body,
    out_type=jax.ShapeDtypeStruct((NC * NACC,), jnp.float32),
    mesh=_sc_mesh,
    scratch_types=[
        pltpu.VMEM_SHARED((NACC,), jnp.float32),
        pltpu.VMEM((KJ, CH), jnp.int32),
        pltpu.VMEM((CH,), jnp.float32),
        pltpu.VMEM((RPS,), jnp.float32),
        pltpu.SemaphoreType.DMA,
    ],
)


NB = 2
CHA = 64
KJA = EP // (NW * CHA)
HK = KJA // 2


def _sc_agg_body(src_hbm, dst_hbm, g_hbm, out_hbm,
                 acc, src_buf, dst_buf, rows0, rows1,
                 sg0, sg1, ss0, ss1):
    c = lax.axis_index("c")
    s = lax.axis_index("s")
    w = c * NS + s

    def zrow(i, carry):
        rows0[i, pl.ds(0, 16)] = jnp.zeros((16,), jnp.float32)
        for b in range(1, D // 16):
            rows0[i, pl.ds(b * 16, 16)] = jnp.zeros((16,), jnp.float32)
        return carry

    lax.fori_loop(0, CHA, zrow, 0)
    for k in range(RPS // CHA):
        pltpu.sync_copy(rows0, acc.at[pl.ds(s * RPS + k * CHA, CHA)])
    rows = (rows0, rows1)
    sgs = (sg0, sg1)
    sss = (ss0, ss1)
    for phase in range(KJA // HK):
        base_j = w * KJA + phase * HK
        pltpu.sync_copy(src_hbm.at[pl.ds(base_j, HK)], src_buf)
        pltpu.sync_copy(dst_hbm.at[pl.ds(base_j, HK)], dst_buf)
        if phase == 0:
            plsc.subcore_barrier()
        for b in range(NB):
            pltpu.async_copy(g_hbm.at[src_buf.at[b]], rows[b], sgs[b])

        def body(jj, carry):
            j0 = jj * NB
            for b in range(NB):
                jb = j0 + b
                pltpu.make_async_copy(g_hbm.at[src_buf.at[jb]], rows[b], sgs[b]).wait()
                pltpu.async_copy(rows[b], acc.at[dst_buf.at[jb]], sss[b], add=True)
            for b in range(NB):
                jb = j0 + b
                pltpu.make_async_copy(rows[b], acc.at[dst_buf.at[jb]], sss[b]).wait()

                @pl.when(jb + NB < HK)
                def _():
                    pltpu.async_copy(g_hbm.at[src_buf.at[jb + NB]], rows[b], sgs[b])

            return carry

        lax.fori_loop(0, HK // NB, body, 0)
    plsc.subcore_barrier()
    for k in range(RPS // CHA):
        base = s * RPS + k * CHA
        pltpu.sync_copy(acc.at[pl.ds(base, CHA)], rows0)
        pltpu.sync_copy(rows0, out_hbm.at[c, pl.ds(base, CHA)])


_agg_call = pl.kernel(
    _sc_agg_body,
    out_type=jax.ShapeDtypeStruct((NC, NACC, D), jnp.float32),
    mesh=_sc_mesh,
    scratch_types=[
        pltpu.VMEM_SHARED((NACC, D), jnp.float32),
        pltpu.VMEM((HK, CHA), jnp.int32),
        pltpu.VMEM((HK, CHA), jnp.int32),
        pltpu.VMEM((CHA, D), jnp.float32),
        pltpu.VMEM((CHA, D), jnp.float32),
        pltpu.SemaphoreType.DMA,
        pltpu.SemaphoreType.DMA,
        pltpu.SemaphoreType.DMA,
        pltpu.SemaphoreType.DMA,
    ],
)


BLK = 1000


def _tc1_body(degp_ref, x_ref, w1_ref, g1_ref):
    deg = degp_ref[0, :, 0:1] + degp_ref[1, :, 0:1] + 1.0
    dis = lax.rsqrt(deg)
    g1_ref[...] = jnp.dot(x_ref[...], w1_ref[...],
                          preferred_element_type=jnp.float32) * dis


def _tc1(degp, x, W1):
    return pl.pallas_call(
        _tc1_body,
        grid=(N // BLK,),
        in_specs=[
            pl.BlockSpec((NC, BLK, 1), lambda i: (0, i, 0)),
            pl.BlockSpec((BLK, D), lambda i: (i, 0)),
            pl.BlockSpec((D, D), lambda i: (0, 0)),
        ],
        out_specs=pl.BlockSpec((BLK, D), lambda i: (i, 0)),
        out_shape=jax.ShapeDtypeStruct((N, D), jnp.float32),
    )(degp, x, W1)


def _tc2_body(degp_ref, ap_ref, g1_ref, w2_ref, b1_ref, g2_ref):
    deg = degp_ref[0, :, 0:1] + degp_ref[1, :, 0:1] + 1.0
    dis = lax.rsqrt(deg)
    h1 = jnp.maximum((ap_ref[0] + ap_ref[1] + g1_ref[...]) * dis + b1_ref[...], 0.0)
    g2_ref[...] = jnp.dot(h1, w2_ref[...],
                          preferred_element_type=jnp.float32) * dis


def _tc2(degp, a1, g1, W2, b1):
    return pl.pallas_call(
        _tc2_body,
        grid=(N // BLK,),
        in_specs=[
            pl.BlockSpec((NC, BLK, 1), lambda i: (0, i, 0)),
            pl.BlockSpec((NC, BLK, D), lambda i: (0, i, 0)),
            pl.BlockSpec((BLK, D), lambda i: (i, 0)),
            pl.BlockSpec((D, D), lambda i: (0, 0)),
            pl.BlockSpec((1, D), lambda i: (0, 0)),
        ],
        out_specs=pl.BlockSpec((BLK, D), lambda i: (i, 0)),
        out_shape=jax.ShapeDtypeStruct((N, D), jnp.float32),
    )(degp, a1, g1, W2, b1)


def _tc3_body(degp_ref, ap_ref, g2_ref, b2_ref, gam_ref, bet_ref, batch_ref,
              wm1_ref, bm1_ref, wm2_ref, bm2_ref, out_ref):
    deg = degp_ref[0, :N, 0:1] + degp_ref[1, :N, 0:1] + 1.0
    dis = lax.rsqrt(deg)
    h2 = (ap_ref[0, :N] + ap_ref[1, :N] + g2_ref[...]) * dis + b2_ref[...]
    mean = jnp.mean(h2, axis=0, keepdims=True)
    var = jnp.mean((h2 - mean) ** 2, axis=0, keepdims=True)
    hb = (h2 - mean) * lax.rsqrt(var + 1e-5) * gam_ref[...] + bet_ref[...]
    ohT = (jax.lax.broadcasted_iota(jnp.int32, (G, N), 0)
           == batch_ref[...]).astype(jnp.float32)
    sums = jnp.dot(ohT, hb, preferred_element_type=jnp.float32)
    counts = jnp.sum(ohT, axis=1, keepdims=True)
    pooled = sums / jnp.maximum(counts, 1.0)
    z = jnp.maximum(jnp.dot(pooled, wm1_ref[...],
                            preferred_element_type=jnp.float32) + bm1_ref[...], 0.0)
    out_ref[...] = jnp.dot(z, wm2_ref[...],
                           preferred_element_type=jnp.float32) + bm2_ref[...]


def _tc3(degp, a2, g2, b2, gamma, beta, batch_row, Wm1, bm1, Wm2, bm2):
    return pl.pallas_call(
        _tc3_body,
        in_specs=[
            pl.BlockSpec((NC, NACC, 1), lambda: (0, 0, 0)),
            pl.BlockSpec((NC, NACC, D), lambda: (0, 0, 0)),
            pl.BlockSpec((N, D), lambda: (0, 0)),
            pl.BlockSpec((1, D), lambda: (0, 0)),
            pl.BlockSpec((1, D), lambda: (0, 0)),
            pl.BlockSpec((1, D), lambda: (0, 0)),
            pl.BlockSpec((1, N), lambda: (0, 0)),
            pl.BlockSpec((D, D), lambda: (0, 0)),
            pl.BlockSpec((1, D), lambda: (0, 0)),
            pl.BlockSpec((D, 1), lambda: (0, 0)),
            pl.BlockSpec((1, 1), lambda: (0, 0)),
        ],
        out_specs=pl.BlockSpec((G, 1), lambda: (0, 0)),
        out_shape=jax.ShapeDtypeStruct((G, 1), jnp.float32),
    )(degp, a2, g2, b2, gamma, beta, batch_row, Wm1, bm1, Wm2, bm2)


def kernel(x, edge_index, batch, W1, b1, W2, b2, gamma, beta, Wm1, bm1, Wm2, bm2):
    src = edge_index[0]
    dst = edge_index[1]
    pad = EP - E
    ar = jnp.arange(pad, dtype=jnp.int32)
    pad_src = ar % N
    pad_dst = N + (ar % (NACC - N))
    srcp = jnp.concatenate([src, pad_src]).reshape(NW * KJ, CH)
    dstp = jnp.concatenate([dst, pad_dst]).reshape(NW * KJ, CH)
    srcp64 = srcp.reshape(NW * KJA, CHA)
    dstp64 = dstp.reshape(NW * KJA, CHA)
    degp = _deg_call(dstp).reshape(NC, NACC, 1)
    g1 = _tc1(degp, x, W1)
    a1 = _agg_call(srcp64, dstp64, g1)
    g2 = _tc2(degp, a1, g1, W2, b1.reshape(1, D))
    a2 = _agg_call(srcp64, dstp64, g2)
    return _tc3(degp, a2, g2, b2.reshape(1, D), gamma.reshape(1, D),
                beta.reshape(1, D), batch.reshape(1, N), Wm1,
                bm1.reshape(1, D), Wm2, bm2.reshape(1, 1))


# speedup vs baseline: 24.1524x; 1.0074x over previous
import jax
import jax.numpy as jnp
from jax import lax
from jax.experimental import pallas as pl
from jax.experimental.pallas import tpu as pltpu
from jax.experimental.pallas import tpu_sc as plsc

N = 10000
E = 320000
D = 128
G = 64

NC = 2
NS = 16
NW = NC * NS
CH = 128
KJ = 80
EP = NW * KJ * CH
NACC = 10240
RPS = NACC // NS
DW = 16

_sc_mesh = plsc.VectorSubcoreMesh(core_axis_name="c", subcore_axis_name="s")


def _sc_deg_body(dst_hbm, out_hbm, acc1d, dst_buf, ones_v, tmp_v, sem):
    c = lax.axis_index("c")
    s = lax.axis_index("s")
    w = c * NS + s

    def ztmp(i, carry):
        tmp_v[pl.ds(i * 16, 16)] = jnp.zeros((16,), jnp.float32)
        return carry

    lax.fori_loop(0, RPS // 16, ztmp, 0)
    pltpu.sync_copy(tmp_v, acc1d.at[pl.ds(s * RPS, RPS)])
    pltpu.sync_copy(dst_hbm.at[pl.ds(w * KJ, KJ)], dst_buf)

    def fill_ones(i, carry):
        ones_v[pl.ds(i * 16, 16)] = jnp.ones((16,), jnp.float32)
        return carry

    lax.fori_loop(0, CH // 16, fill_ones, 0)
    plsc.subcore_barrier()

    def body(jj, carry):
        j0 = jj * 8
        for b in range(8):
            pltpu.async_copy(ones_v, acc1d.at[dst_buf.at[j0 + b]], sem, add=True)
        for b in range(8):
            pltpu.make_async_copy(ones_v, acc1d.at[dst_buf.at[j0 + b]], sem).wait()
        return carry

    lax.fori_loop(0, KJ // 8, body, 0)
    plsc.subcore_barrier()
    pltpu.sync_copy(acc1d.at[pl.ds(s * RPS, RPS)], tmp_v)
    pltpu.sync_copy(tmp_v, out_hbm.at[pl.ds(c * NACC + s * RPS, RPS)])


_deg_call = pl.kernel(
    _sc_deg_body,
    out_type=jax.ShapeDtypeStruct((NC * NACC,), jnp.float32),
    mesh=_sc_mesh,
    scratch_types=[
        pltpu.VMEM_SHARED((NACC,), jnp.float32),
        pltpu.VMEM((KJ, CH), jnp.int32),
        pltpu.VMEM((CH,), jnp.float32),
        pltpu.VMEM((RPS,), jnp.float32),
        pltpu.SemaphoreType.DMA,
    ],
)


NB = 2
CHA = 64
KJA = EP // (NW * CHA)
HK = KJA // 2


def _sc_agg_body(src_hbm, dst_hbm, g_hbm, out_hbm,
                 acc, src_buf, dst_buf, rows0, rows1,
                 sg0, sg1, ss0, ss1):
    c = lax.axis_index("c")
    s = lax.axis_index("s")
    w = c * NS + s

    def zrow(i, carry):
        rows0[i, pl.ds(0, 16)] = jnp.zeros((16,), jnp.float32)
        for b in range(1, D // 16):
            rows0[i, pl.ds(b * 16, 16)] = jnp.zeros((16,), jnp.float32)
        return carry

    lax.fori_loop(0, CHA, zrow, 0)
    for k in range(RPS // CHA):
        pltpu.sync_copy(rows0, acc.at[pl.ds(s * RPS + k * CHA, CHA)])
    rows = (rows0, rows1)
    sgs = (sg0, sg1)
    sss = (ss0, ss1)
    for phase in range(KJA // HK):
        base_j = w * KJA + phase * HK
        pltpu.sync_copy(src_hbm.at[pl.ds(base_j, HK)], src_buf)
        pltpu.sync_copy(dst_hbm.at[pl.ds(base_j, HK)], dst_buf)
        if phase == 0:
            plsc.subcore_barrier()
        for b in range(NB):
            pltpu.async_copy(g_hbm.at[src_buf.at[b]], rows[b], sgs[b])

        def body(jj, carry):
            j0 = jj * NB
            for b in range(NB):
                jb = j0 + b
                pltpu.make_async_copy(g_hbm.at[src_buf.at[jb]], rows[b], sgs[b]).wait()
                pltpu.async_copy(rows[b], acc.at[dst_buf.at[jb]], sss[b], add=True)
            for b in range(NB):
                jb = j0 + b
                pltpu.make_async_copy(rows[b], acc.at[dst_buf.at[jb]], sss[b]).wait()

                @pl.when(jb + NB < HK)
                def _():
                    pltpu.async_copy(g_hbm.at[src_buf.at[jb + NB]], rows[b], sgs[b])

            return carry

        lax.fori_loop(0, HK // NB, body, 0)
    plsc.subcore_barrier()
    nko = RPS // CHA
    for k in range(nko):
        b = k % 2
        if k >= 2:
            pltpu.make_async_copy(rows[b], out_hbm.at[c, pl.ds(0, CHA)], sgs[b]).wait()
        base = s * RPS + k * CHA
        pltpu.sync_copy(acc.at[pl.ds(base, CHA)], rows[b])
        pltpu.async_copy(rows[b], out_hbm.at[c, pl.ds(base, CHA)], sgs[b])
    for k in (nko - 2, nko - 1):
        pltpu.make_async_copy(rows[k % 2], out_hbm.at[c, pl.ds(0, CHA)], sgs[k % 2]).wait()


_agg_call = pl.kernel(
    _sc_agg_body,
    out_type=jax.ShapeDtypeStruct((NC, NACC, D), jnp.float32),
    mesh=_sc_mesh,
    scratch_types=[
        pltpu.VMEM_SHARED((NACC, D), jnp.float32),
        pltpu.VMEM((HK, CHA), jnp.int32),
        pltpu.VMEM((HK, CHA), jnp.int32),
        pltpu.VMEM((CHA, D), jnp.float32),
        pltpu.VMEM((CHA, D), jnp.float32),
        pltpu.SemaphoreType.DMA,
        pltpu.SemaphoreType.DMA,
        pltpu.SemaphoreType.DMA,
        pltpu.SemaphoreType.DMA,
    ],
)


BLK = 1000


def _tc1_body(degp_ref, x_ref, w1_ref, g1_ref):
    deg = degp_ref[0, :, 0:1] + degp_ref[1, :, 0:1] + 1.0
    dis = lax.rsqrt(deg)
    g1_ref[...] = jnp.dot(x_ref[...], w1_ref[...],
                          preferred_element_type=jnp.float32) * dis


def _tc1(degp, x, W1):
    return pl.pallas_call(
        _tc1_body,
        grid=(N // BLK,),
        in_specs=[
            pl.BlockSpec((NC, BLK, 1), lambda i: (0, i, 0)),
            pl.BlockSpec((BLK, D), lambda i: (i, 0)),
            pl.BlockSpec((D, D), lambda i: (0, 0)),
        ],
        out_specs=pl.BlockSpec((BLK, D), lambda i: (i, 0)),
        out_shape=jax.ShapeDtypeStruct((N, D), jnp.float32),
    )(degp, x, W1)


def _tc2_body(degp_ref, ap_ref, g1_ref, w2_ref, b1_ref, g2_ref):
    deg = degp_ref[0, :, 0:1] + degp_ref[1, :, 0:1] + 1.0
    dis = lax.rsqrt(deg)
    h1 = jnp.maximum((ap_ref[0] + ap_ref[1] + g1_ref[...]) * dis + b1_ref[...], 0.0)
    g2_ref[...] = jnp.dot(h1, w2_ref[...],
                          preferred_element_type=jnp.float32) * dis


def _tc2(degp, a1, g1, W2, b1):
    return pl.pallas_call(
        _tc2_body,
        grid=(N // BLK,),
        in_specs=[
            pl.BlockSpec((NC, BLK, 1), lambda i: (0, i, 0)),
            pl.BlockSpec((NC, BLK, D), lambda i: (0, i, 0)),
            pl.BlockSpec((BLK, D), lambda i: (i, 0)),
            pl.BlockSpec((D, D), lambda i: (0, 0)),
            pl.BlockSpec((1, D), lambda i: (0, 0)),
        ],
        out_specs=pl.BlockSpec((BLK, D), lambda i: (i, 0)),
        out_shape=jax.ShapeDtypeStruct((N, D), jnp.float32),
    )(degp, a1, g1, W2, b1)


def _tc3_body(degp_ref, ap_ref, g2_ref, b2_ref, gam_ref, bet_ref, batch_ref,
              wm1_ref, bm1_ref, wm2_ref, bm2_ref, out_ref):
    deg = degp_ref[0, :N, 0:1] + degp_ref[1, :N, 0:1] + 1.0
    dis = lax.rsqrt(deg)
    h2 = (ap_ref[0, :N] + ap_ref[1, :N] + g2_ref[...]) * dis + b2_ref[...]
    mean = jnp.mean(h2, axis=0, keepdims=True)
    var = jnp.mean((h2 - mean) ** 2, axis=0, keepdims=True)
    hb = (h2 - mean) * lax.rsqrt(var + 1e-5) * gam_ref[...] + bet_ref[...]
    ohT = (jax.lax.broadcasted_iota(jnp.int32, (G, N), 0)
           == batch_ref[...]).astype(jnp.float32)
    sums = jnp.dot(ohT, hb, preferred_element_type=jnp.float32)
    counts = jnp.sum(ohT, axis=1, keepdims=True)
    pooled = sums / jnp.maximum(counts, 1.0)
    z = jnp.maximum(jnp.dot(pooled, wm1_ref[...],
                            preferred_element_type=jnp.float32) + bm1_ref[...], 0.0)
    out_ref[...] = jnp.dot(z, wm2_ref[...],
                           preferred_element_type=jnp.float32) + bm2_ref[...]


def _tc3(degp, a2, g2, b2, gamma, beta, batch_row, Wm1, bm1, Wm2, bm2):
    return pl.pallas_call(
        _tc3_body,
        in_specs=[
            pl.BlockSpec((NC, NACC, 1), lambda: (0, 0, 0)),
            pl.BlockSpec((NC, NACC, D), lambda: (0, 0, 0)),
            pl.BlockSpec((N, D), lambda: (0, 0)),
            pl.BlockSpec((1, D), lambda: (0, 0)),
            pl.BlockSpec((1, D), lambda: (0, 0)),
            pl.BlockSpec((1, D), lambda: (0, 0)),
            pl.BlockSpec((1, N), lambda: (0, 0)),
            pl.BlockSpec((D, D), lambda: (0, 0)),
            pl.BlockSpec((1, D), lambda: (0, 0)),
            pl.BlockSpec((D, 1), lambda: (0, 0)),
            pl.BlockSpec((1, 1), lambda: (0, 0)),
        ],
        out_specs=pl.BlockSpec((G, 1), lambda: (0, 0)),
        out_shape=jax.ShapeDtypeStruct((G, 1), jnp.float32),
    )(degp, a2, g2, b2, gamma, beta, batch_row, Wm1, bm1, Wm2, bm2)


def kernel(x, edge_index, batch, W1, b1, W2, b2, gamma, beta, Wm1, bm1, Wm2, bm2):
    src = edge_index[0]
    dst = edge_index[1]
    pad = EP - E
    ar = jnp.arange(pad, dtype=jnp.int32)
    pad_src = ar % N
    pad_dst = N + (ar % (NACC - N))
    srcp = jnp.concatenate([src, pad_src]).reshape(NW * KJ, CH)
    dstp = jnp.concatenate([dst, pad_dst]).reshape(NW * KJ, CH)
    srcp64 = srcp.reshape(NW * KJA, CHA)
    dstp64 = dstp.reshape(NW * KJA, CHA)
    degp = _deg_call(dstp).reshape(NC, NACC, 1)
    g1 = _tc1(degp, x, W1)
    a1 = _agg_call(srcp64, dstp64, g1)
    g2 = _tc2(degp, a1, g1, W2, b1.reshape(1, D))
    a2 = _agg_call(srcp64, dstp64, g2)
    return _tc3(degp, a2, g2, b2.reshape(1, D), gamma.reshape(1, D),
                beta.reshape(1, D), batch.reshape(1, N), Wm1,
                bm1.reshape(1, D), Wm2, bm2.reshape(1, 1))
